# Initial kernel scaffold; baseline (speedup 1.0000x reference)
import jax
import jax.numpy as jnp
from jax.experimental import pallas as pl


def kernel(input_ids, W_word, W_pos, W_tok, gamma, beta):
    raise NotImplementedError("write your pallas kernel here")


# speedup vs baseline: 7.5034x; 7.5034x over previous
import jax
import jax.numpy as jnp
from jax.experimental import pallas as pl

_NUM_TASKS = 4
_HIDDEN = 768
_EPS = 1e-12


def _tc_body(ids_ref, ww_ref, wp_ref, wt_ref, g_ref, b_ref, out_ref):
    s = ww_ref[...] + wt_ref[...] + wp_ref[...]
    mean = jnp.mean(s, axis=-1, keepdims=True)
    var = jnp.mean(jnp.square(s - mean), axis=-1, keepdims=True)
    comb = (s - mean) * jax.lax.rsqrt(var + _EPS) * g_ref[...] + b_ref[...]

    ids = ids_ref[0]
    onehot_t = (jax.lax.broadcasted_iota(jnp.int32, (_NUM_TASKS, ids.shape[1]), 0)
                == ids).astype(jnp.float32)
    out_ref[...] = jax.lax.dot_general(
        onehot_t, comb, (((0,), (0,)), ((), ())),
        preferred_element_type=jnp.float32)


def kernel(input_ids, W_word, W_pos, W_tok, gamma, beta):
    batch, l = input_ids.shape
    n = batch * l
    block = 2048
    grid = n // block
    ids3 = input_ids.reshape(grid, 1, block).astype(jnp.int32)
    g2 = gamma.reshape(1, _HIDDEN)
    b2 = beta.reshape(1, _HIDDEN)

    out = pl.pallas_call(
        _tc_body,
        grid=(grid,),
        in_specs=[
            pl.BlockSpec((1, 1, block), lambda i: (i, 0, 0)),
            pl.BlockSpec((_NUM_TASKS, _HIDDEN), lambda i: (0, 0)),
            pl.BlockSpec((_NUM_TASKS, _HIDDEN), lambda i: (0, 0)),
            pl.BlockSpec((_NUM_TASKS, _HIDDEN), lambda i: (0, 0)),
            pl.BlockSpec((1, _HIDDEN), lambda i: (0, 0)),
            pl.BlockSpec((1, _HIDDEN), lambda i: (0, 0)),
        ],
        out_specs=pl.BlockSpec((block, _HIDDEN), lambda i: (i, 0)),
        out_shape=jax.ShapeDtypeStruct((n, _HIDDEN), jnp.float32),
    )(ids3, W_word, W_pos, W_tok, g2, b2)
    return out.reshape(batch, l, _HIDDEN)
